# baseline (speedup 1.0000x reference)
import functools

import jax
import jax.numpy as jnp
from jax import lax
from jax.experimental import pallas as pl
from jax.experimental.pallas import tpu as pltpu
from jax.experimental.pallas import tpu_sc as plsc

HID = 2048
NE = 16
NT = 16384
TOPK = 2
TM = 1024

NC = 2
NS = 16
NW = NC * NS
TPW = NT // NW
LANES = 16
NG = TPW // LANES


def _logits_body(x_ref, wt_ref, b_ref, out_ref, outt_ref):
    acc = lax.dot_general(
        x_ref[...], wt_ref[...], (((1,), (0,)), ((), ())),
        preferred_element_type=jnp.float32)
    acc = acc + b_ref[...]
    out_ref[...] = acc
    outt_ref[...] = acc.T


def _compute_logits(x, Wt, b2):
    return pl.pallas_call(
        _logits_body,
        grid=(NT // TM,),
        in_specs=[
            pl.BlockSpec((TM, HID), lambda i: (i, 0)),
            pl.BlockSpec((HID, NE), lambda i: (0, 0)),
            pl.BlockSpec((1, NE), lambda i: (0, 0)),
        ],
        out_specs=[
            pl.BlockSpec((TM, NE), lambda i: (i, 0)),
            pl.BlockSpec((NE, TM), lambda i: (0, i)),
        ],
        out_shape=[
            jax.ShapeDtypeStruct((NT, NE), jnp.float32),
            jax.ShapeDtypeStruct((NE, NT), jnp.float32),
        ],
        compiler_params=pltpu.CompilerParams(
            dimension_semantics=("arbitrary",)),
    )(x, Wt, b2)


def _combine(av, ai, bv, bi):
    take = bv > av
    return jnp.where(take, bv, av), jnp.where(take, bi, ai)


def _tree_max(vals, idxs):
    while len(vals) > 1:
        nv, ni = [], []
        for j in range(0, len(vals), 2):
            v, i = _combine(vals[j], idxs[j], vals[j + 1], idxs[j + 1])
            nv.append(v)
            ni.append(i)
        vals, idxs = nv, ni
    return vals[0], idxs[0]


@functools.partial(
    pl.kernel,
    mesh=plsc.VectorSubcoreMesh(core_axis_name="c", subcore_axis_name="s"),
    out_type=[
        jax.ShapeDtypeStruct((NT * TOPK,), jnp.float32),
        jax.ShapeDtypeStruct((NT * TOPK,), jnp.int32),
        jax.ShapeDtypeStruct((NE * TOPK, NT), jnp.int32),
    ],
    scratch_types=[
        pltpu.VMEM((NE, TPW), jnp.float32),
        pltpu.VMEM((TPW * TOPK,), jnp.float32),
        pltpu.VMEM((TPW * TOPK,), jnp.int32),
        pltpu.VMEM((NE * TOPK, TPW), jnp.int32),
    ],
    compiler_params=pltpu.CompilerParams(needs_layout_passes=False),
)
def _route(lgt_hbm, w_hbm, i_hbm, m_hbm, lgt_v, w_v, i_v, m_v):
    c = lax.axis_index("c")
    s = lax.axis_index("s")
    wid = s * NC + c
    base = wid * TPW
    pltpu.sync_copy(lgt_hbm.at[:, pl.ds(base, TPW)], lgt_v)

    lanes = lax.iota(jnp.int32, LANES)

    def group(g, carry):
        t0 = g * LANES
        tok = t0 + lanes
        vs = [lgt_v[e, pl.ds(t0, LANES)] for e in range(NE)]
        eidx = [jnp.full((LANES,), e, jnp.int32) for e in range(NE)]
        m1, i1 = _tree_max(list(vs), list(eidx))
        neg = jnp.full((LANES,), -jnp.inf, jnp.float32)
        vs2 = [jnp.where(i1 == e, neg, vs[e]) for e in range(NE)]
        m2, i2 = _tree_max(vs2, list(eidx))
        r = jnp.exp(m2 - m1)
        den = 1.0 + r
        w1 = 1.0 / den
        w2 = r / den
        plsc.store_scatter(w_v, [tok * TOPK], w1)
        plsc.store_scatter(w_v, [tok * TOPK + 1], w2)
        plsc.store_scatter(i_v, [tok * TOPK], i1)
        plsc.store_scatter(i_v, [tok * TOPK + 1], i2)
        one = jnp.full((LANES,), 1, jnp.int32)
        zero = jnp.zeros((LANES,), jnp.int32)
        for e in range(NE):
            m_v[e * TOPK, pl.ds(t0, LANES)] = jnp.where(i1 == e, one, zero)
            m_v[e * TOPK + 1, pl.ds(t0, LANES)] = jnp.where(i2 == e, one, zero)
        return carry

    lax.fori_loop(0, NG, group, 0)
    pltpu.sync_copy(w_v, w_hbm.at[pl.ds(base * TOPK, TPW * TOPK)])
    pltpu.sync_copy(i_v, i_hbm.at[pl.ds(base * TOPK, TPW * TOPK)])
    pltpu.sync_copy(m_v, m_hbm.at[:, pl.ds(base, TPW)])


def kernel(x, W, b):
    logits, logitsT = _compute_logits(x, W.T, b.reshape(1, NE))
    wflat, iflat, mrows = _route(logitsT)
    return (
        logits,
        wflat.reshape(NT, TOPK),
        iflat.reshape(NT, TOPK),
        mrows.reshape(NE, TOPK, NT),
    )
